# baseline (speedup 1.0000x reference)
import functools

import jax
import jax.numpy as jnp
from jax import lax
from jax.experimental import pallas as pl
from jax.experimental.pallas import tpu as pltpu
from jax.experimental.pallas import tpu_sc as plsc

N = 10000
NP = 10240
E = 320000
D = 128
H = 128
C = 40
CP = 48

NC = 2
NS = 16
NW = NC * NS
K = 80
EPW = E // NW
RPW = NP // NS

BM = 1024


NBLK = E // K
NBW = NBLK // NW
_MESH = plsc.VectorSubcoreMesh(core_axis_name="c", subcore_axis_name="s")
_SC_PARAMS = pltpu.CompilerParams(use_tc_tiling_on_sc=False)


def _sc_segment_sum(table, ei, width, nbuf):
    nt = (NBW // nbuf) * nbuf
    zeros = jnp.zeros((NP, width), jnp.float32)

    @functools.partial(
        pl.kernel,
        mesh=_MESH,
        compiler_params=_SC_PARAMS,
        out_type=jax.ShapeDtypeStruct((NC, NP, width), jnp.float32),
        scratch_types=[
            pltpu.VMEM((NBW * K,), jnp.int32),
            pltpu.VMEM((NBW * K,), jnp.int32),
            [pltpu.VMEM((K, width), jnp.float32)] * nbuf,
            pltpu.VMEM_SHARED((NP, width), jnp.float32),
            [pltpu.SemaphoreType.DMA] * nbuf,
        ],
    )
    def k(zeros_hbm, table_hbm, ei_hbm, out_hbm,
          sidx, didx, bufs, acc, sems):
        c = lax.axis_index("c")
        s = lax.axis_index("s")
        wid = c * NS + s
        r0 = s * RPW
        pltpu.sync_copy(zeros_hbm.at[pl.ds(r0, RPW)], acc.at[pl.ds(r0, RPW)])
        pltpu.sync_copy(ei_hbm.at[0, pl.ds(wid * NBW * K, NBW * K)], sidx)
        pltpu.sync_copy(ei_hbm.at[1, pl.ds(wid * NBW * K, NBW * K)], didx)
        plsc.subcore_barrier()

        for b in range(nbuf):
            pltpu.async_copy(table_hbm.at[sidx.at[pl.ds(b * K, K)]], bufs[b], sems[b])

        @pl.loop(0, nt, step=nbuf)
        def _(j):
            for b in range(nbuf):
                pltpu.make_async_copy(
                    table_hbm.at[sidx.at[pl.ds((j + b) * K, K)]], bufs[b], sems[b]).wait()
                pltpu.sync_copy(
                    bufs[b], acc.at[didx.at[pl.ds((j + b) * K, K)]], add=True)

                @pl.when(j + b + nbuf < NBW)
                def _():
                    pltpu.async_copy(
                        table_hbm.at[sidx.at[pl.ds((j + b + nbuf) * K, K)]],
                        bufs[b], sems[b])

        for t in range(nt, NBW):
            b = t % nbuf
            pltpu.make_async_copy(
                table_hbm.at[sidx.at[pl.ds(t * K, K)]], bufs[b], sems[b]).wait()
            pltpu.sync_copy(bufs[b], acc.at[didx.at[pl.ds(t * K, K)]], add=True)

        plsc.subcore_barrier()
        pltpu.sync_copy(acc.at[pl.ds(r0, RPW)], out_hbm.at[c, pl.ds(r0, RPW)])

    return k(zeros, table, ei)


HH = H // 2
NBS = NBLK // NS
NCHUNK = 2
CBW = NBS // NCHUNK
NB1 = 12


def _sc_agg_colsplit(table2, ei):
    zeros = jnp.zeros((NP, HH), jnp.float32)

    @functools.partial(
        pl.kernel,
        mesh=_MESH,
        compiler_params=_SC_PARAMS,
        out_type=jax.ShapeDtypeStruct((NC, NP, HH), jnp.float32),
        scratch_types=[
            pltpu.VMEM((CBW * K,), jnp.int32),
            pltpu.VMEM((CBW * K,), jnp.int32),
            [pltpu.VMEM((K, HH), jnp.float32)] * NB1,
            pltpu.VMEM_SHARED((NP, HH), jnp.float32),
            [pltpu.SemaphoreType.DMA] * NB1,
        ],
    )
    def k(zeros_hbm, table_hbm, ei_hbm, out_hbm,
          sidx, didx, bufs, acc, sems):
        c = lax.axis_index("c")
        s = lax.axis_index("s")
        r0 = s * RPW
        pltpu.sync_copy(zeros_hbm.at[pl.ds(r0, RPW)], acc.at[pl.ds(r0, RPW)])
        plsc.subcore_barrier()
        half = table_hbm.at[c]

        for chunk in range(NCHUNK):
            base = (s * NBS + chunk * CBW) * K
            pltpu.sync_copy(ei_hbm.at[0, pl.ds(base, CBW * K)], sidx)
            pltpu.sync_copy(ei_hbm.at[1, pl.ds(base, CBW * K)], didx)

            for b in range(NB1):
                pltpu.async_copy(half.at[sidx.at[pl.ds(b * K, K)]], bufs[b], sems[b])

            @pl.loop(0, (CBW // NB1) * NB1, step=NB1)
            def _(j):
                for b in range(NB1):
                    pltpu.make_async_copy(
                        half.at[sidx.at[pl.ds((j + b) * K, K)]], bufs[b], sems[b]).wait()
                    pltpu.sync_copy(
                        bufs[b], acc.at[didx.at[pl.ds((j + b) * K, K)]], add=True)

                    @pl.when(j + b + NB1 < CBW)
                    def _():
                        pltpu.async_copy(
                            half.at[sidx.at[pl.ds((j + b + NB1) * K, K)]],
                            bufs[b], sems[b])

            for t in range((CBW // NB1) * NB1, CBW):
                b = t % NB1
                pltpu.make_async_copy(
                    half.at[sidx.at[pl.ds(t * K, K)]], bufs[b], sems[b]).wait()
                pltpu.sync_copy(bufs[b], acc.at[didx.at[pl.ds(t * K, K)]], add=True)

        plsc.subcore_barrier()
        pltpu.sync_copy(acc.at[pl.ds(r0, RPW)], out_hbm.at[c, pl.ds(r0, RPW)])

    return k(zeros, table2, ei)


def _sc_degree(ei):
    zeros = jnp.zeros((NP, 16), jnp.float32)
    ones = jnp.ones((K, 16), jnp.float32)

    @functools.partial(
        pl.kernel,
        mesh=_MESH,
        compiler_params=_SC_PARAMS,
        out_type=jax.ShapeDtypeStruct((NC, NP, 16), jnp.float32),
        scratch_types=[
            pltpu.VMEM((NBW * K,), jnp.int32),
            pltpu.VMEM((K, 16), jnp.float32),
            pltpu.VMEM_SHARED((NP, 16), jnp.float32),
            pltpu.SemaphoreType.DMA,
        ],
    )
    def k(zeros_hbm, ones_hbm, ei_hbm, out_hbm, idx, ones_v, acc, sem):
        c = lax.axis_index("c")
        s = lax.axis_index("s")
        wid = c * NS + s
        r0 = s * (NP // NS)
        pltpu.sync_copy(zeros_hbm.at[pl.ds(r0, NP // NS)], acc.at[pl.ds(r0, NP // NS)])
        pltpu.sync_copy(ones_hbm, ones_v)
        pltpu.sync_copy(ei_hbm.at[1, pl.ds(wid * NBW * K, NBW * K)], idx)
        plsc.subcore_barrier()

        @pl.loop(0, NBW)
        def _(j):
            pltpu.async_copy(ones_v, acc.at[idx.at[pl.ds(j * K, K)]], sem, add=True)

        @pl.loop(0, NBW)
        def _(j):
            pltpu.make_async_copy(ones_v, acc.at[idx.at[pl.ds(j * K, K)]], sem).wait()

        plsc.subcore_barrier()
        pltpu.sync_copy(acc.at[pl.ds(r0, NP // NS)], out_hbm.at[c, pl.ds(r0, NP // NS)])

    return k(zeros, ones, ei)


def _dinv_block(deg_ref):
    deg = deg_ref[0] + deg_ref[1]
    return lax.rsqrt(deg[:, 0:1] + 1.0)


def _l1_body(deg_ref, x_ref, w_ref, o_ref):
    h = jnp.dot(x_ref[...], w_ref[...], preferred_element_type=jnp.float32)
    hs = h * _dinv_block(deg_ref)
    o_ref[0] = hs[:, :HH]
    o_ref[1] = hs[:, HH:]


def _l2_body(deg_ref, a_ref, hs_ref, b_ref, w_ref, o_ref):
    dinv = _dinv_block(deg_ref)
    t = a_ref[...] + hs_ref[...]
    z = jnp.concatenate([t[0], t[1]], axis=1) * dinv + b_ref[...]
    o1 = jnp.maximum(z, 0.0)
    h2 = jnp.dot(o1, w_ref[...], preferred_element_type=jnp.float32)
    o_ref[...] = h2 * dinv


def _out_body(deg_ref, a_ref, hs_ref, b_ref, o_ref):
    dinv = _dinv_block(deg_ref)
    z = (a_ref[0] + a_ref[1] + hs_ref[...]) * dinv + b_ref[...]
    m = jnp.max(z, axis=1, keepdims=True)
    lse = jnp.log(jnp.sum(jnp.exp(z - m), axis=1, keepdims=True)) + m
    o_ref[...] = (z - lse)[:, :C]


def _deg_spec():
    return pl.BlockSpec((2, BM, 16), lambda i: (0, i, 0))


def _row_spec(width):
    return pl.BlockSpec((BM, width), lambda i: (i, 0))


def _agg_spec(width):
    return pl.BlockSpec((2, BM, width), lambda i: (0, i, 0))


def _full_spec(shape):
    return pl.BlockSpec(shape, lambda i: (0,) * len(shape))


def kernel(x, edge_index, W1, b1, W2, b2):
    ei = edge_index.astype(jnp.int32)

    degp = _sc_degree(ei)

    grid = (NP // BM,)
    hs1 = pl.pallas_call(
        _l1_body,
        grid=grid,
        in_specs=[_deg_spec(), _row_spec(D), _full_spec((D, H))],
        out_specs=_agg_spec(HH),
        out_shape=jax.ShapeDtypeStruct((2, NP, HH), jnp.float32),
    )(degp, x, W1)

    a1 = _sc_agg_colsplit(hs1, ei)

    W2p = jnp.pad(W2, ((0, 0), (0, CP - C)))
    b1r = b1.reshape(1, H)
    hs2 = pl.pallas_call(
        _l2_body,
        grid=grid,
        in_specs=[_deg_spec(), _agg_spec(HH), _agg_spec(HH),
                  _full_spec((1, H)), _full_spec((H, CP))],
        out_specs=_row_spec(CP),
        out_shape=jax.ShapeDtypeStruct((NP, CP), jnp.float32),
    )(degp, a1, hs1, b1r, W2p)

    a2 = _sc_segment_sum(hs2, ei, CP, 8)

    b2p = jnp.concatenate([b2, jnp.full((CP - C,), -1e30, jnp.float32)])
    b2r = b2p.reshape(1, CP)
    out = pl.pallas_call(
        _out_body,
        grid=grid,
        in_specs=[_deg_spec(), _agg_spec(CP), _row_spec(CP),
                  _full_spec((1, CP))],
        out_specs=_row_spec(C),
        out_shape=jax.ShapeDtypeStruct((NP, C), jnp.float32),
    )(degp, a2, hs2, b2r)
    return out[:N]


# speedup vs baseline: 1.0262x; 1.0262x over previous
import functools

import jax
import jax.numpy as jnp
from jax import lax
from jax.experimental import pallas as pl
from jax.experimental.pallas import tpu as pltpu
from jax.experimental.pallas import tpu_sc as plsc

N = 10000
NP = 10240
E = 320000
D = 128
H = 128
C = 40
CP = 48

NC = 2
NS = 16
NW = NC * NS
K = 80
EPW = E // NW
RPW = NP // NS

BM = 2560


NBLK = E // K
NBW = NBLK // NW
_MESH = plsc.VectorSubcoreMesh(core_axis_name="c", subcore_axis_name="s")
_SC_PARAMS = pltpu.CompilerParams(use_tc_tiling_on_sc=False)


def _sc_segment_sum(table, ei, width, nbuf):
    nt = (NBW // nbuf) * nbuf
    zeros = jnp.zeros((NP, width), jnp.float32)

    @functools.partial(
        pl.kernel,
        mesh=_MESH,
        compiler_params=_SC_PARAMS,
        out_type=jax.ShapeDtypeStruct((NC, NP, width), jnp.float32),
        scratch_types=[
            pltpu.VMEM((NBW * K,), jnp.int32),
            pltpu.VMEM((NBW * K,), jnp.int32),
            [pltpu.VMEM((K, width), jnp.float32)] * nbuf,
            pltpu.VMEM_SHARED((NP, width), jnp.float32),
            [pltpu.SemaphoreType.DMA] * nbuf,
        ],
    )
    def k(zeros_hbm, table_hbm, ei_hbm, out_hbm,
          sidx, didx, bufs, acc, sems):
        c = lax.axis_index("c")
        s = lax.axis_index("s")
        wid = c * NS + s
        r0 = s * RPW
        pltpu.sync_copy(zeros_hbm.at[pl.ds(r0, RPW)], acc.at[pl.ds(r0, RPW)])
        pltpu.sync_copy(ei_hbm.at[0, pl.ds(wid * NBW * K, NBW * K)], sidx)
        pltpu.sync_copy(ei_hbm.at[1, pl.ds(wid * NBW * K, NBW * K)], didx)
        plsc.subcore_barrier()

        for b in range(nbuf):
            pltpu.async_copy(table_hbm.at[sidx.at[pl.ds(b * K, K)]], bufs[b], sems[b])

        @pl.loop(0, nt, step=nbuf)
        def _(j):
            for b in range(nbuf):
                pltpu.make_async_copy(
                    table_hbm.at[sidx.at[pl.ds((j + b) * K, K)]], bufs[b], sems[b]).wait()
                pltpu.sync_copy(
                    bufs[b], acc.at[didx.at[pl.ds((j + b) * K, K)]], add=True)

                @pl.when(j + b + nbuf < NBW)
                def _():
                    pltpu.async_copy(
                        table_hbm.at[sidx.at[pl.ds((j + b + nbuf) * K, K)]],
                        bufs[b], sems[b])

        for t in range(nt, NBW):
            b = t % nbuf
            pltpu.make_async_copy(
                table_hbm.at[sidx.at[pl.ds(t * K, K)]], bufs[b], sems[b]).wait()
            pltpu.sync_copy(bufs[b], acc.at[didx.at[pl.ds(t * K, K)]], add=True)

        plsc.subcore_barrier()
        pltpu.sync_copy(acc.at[pl.ds(r0, RPW)], out_hbm.at[c, pl.ds(r0, RPW)])

    return k(zeros, table, ei)


HH = H // 2
NBS = NBLK // NS
NCHUNK = 2
CBW = NBS // NCHUNK
NB1 = 12


def _sc_agg_colsplit(table2, ei):
    zeros = jnp.zeros((NP, HH), jnp.float32)

    @functools.partial(
        pl.kernel,
        mesh=_MESH,
        compiler_params=_SC_PARAMS,
        out_type=jax.ShapeDtypeStruct((NC, NP, HH), jnp.float32),
        scratch_types=[
            pltpu.VMEM((CBW * K,), jnp.int32),
            pltpu.VMEM((CBW * K,), jnp.int32),
            [pltpu.VMEM((K, HH), jnp.float32)] * NB1,
            pltpu.VMEM_SHARED((NP, HH), jnp.float32),
            [pltpu.SemaphoreType.DMA] * NB1,
        ],
    )
    def k(zeros_hbm, table_hbm, ei_hbm, out_hbm,
          sidx, didx, bufs, acc, sems):
        c = lax.axis_index("c")
        s = lax.axis_index("s")
        r0 = s * RPW
        pltpu.sync_copy(zeros_hbm.at[pl.ds(r0, RPW)], acc.at[pl.ds(r0, RPW)])
        plsc.subcore_barrier()
        half = table_hbm.at[c]

        for chunk in range(NCHUNK):
            base = (s * NBS + chunk * CBW) * K
            pltpu.sync_copy(ei_hbm.at[0, pl.ds(base, CBW * K)], sidx)
            pltpu.sync_copy(ei_hbm.at[1, pl.ds(base, CBW * K)], didx)

            for b in range(NB1):
                pltpu.async_copy(half.at[sidx.at[pl.ds(b * K, K)]], bufs[b], sems[b])

            @pl.loop(0, (CBW // NB1) * NB1, step=NB1)
            def _(j):
                for b in range(NB1):
                    pltpu.make_async_copy(
                        half.at[sidx.at[pl.ds((j + b) * K, K)]], bufs[b], sems[b]).wait()
                    pltpu.sync_copy(
                        bufs[b], acc.at[didx.at[pl.ds((j + b) * K, K)]], add=True)

                    @pl.when(j + b + NB1 < CBW)
                    def _():
                        pltpu.async_copy(
                            half.at[sidx.at[pl.ds((j + b + NB1) * K, K)]],
                            bufs[b], sems[b])

            for t in range((CBW // NB1) * NB1, CBW):
                b = t % NB1
                pltpu.make_async_copy(
                    half.at[sidx.at[pl.ds(t * K, K)]], bufs[b], sems[b]).wait()
                pltpu.sync_copy(bufs[b], acc.at[didx.at[pl.ds(t * K, K)]], add=True)

        plsc.subcore_barrier()
        pltpu.sync_copy(acc.at[pl.ds(r0, RPW)], out_hbm.at[c, pl.ds(r0, RPW)])

    return k(zeros, table2, ei)


def _sc_degree(ei):
    zeros = jnp.zeros((NP, 16), jnp.float32)
    ones = jnp.ones((K, 16), jnp.float32)

    @functools.partial(
        pl.kernel,
        mesh=_MESH,
        compiler_params=_SC_PARAMS,
        out_type=jax.ShapeDtypeStruct((NC, NP, 16), jnp.float32),
        scratch_types=[
            pltpu.VMEM((NBW * K,), jnp.int32),
            pltpu.VMEM((K, 16), jnp.float32),
            pltpu.VMEM_SHARED((NP, 16), jnp.float32),
            pltpu.SemaphoreType.DMA,
        ],
    )
    def k(zeros_hbm, ones_hbm, ei_hbm, out_hbm, idx, ones_v, acc, sem):
        c = lax.axis_index("c")
        s = lax.axis_index("s")
        wid = c * NS + s
        r0 = s * (NP // NS)
        pltpu.sync_copy(zeros_hbm.at[pl.ds(r0, NP // NS)], acc.at[pl.ds(r0, NP // NS)])
        pltpu.sync_copy(ones_hbm, ones_v)
        pltpu.sync_copy(ei_hbm.at[1, pl.ds(wid * NBW * K, NBW * K)], idx)
        plsc.subcore_barrier()

        @pl.loop(0, NBW)
        def _(j):
            pltpu.async_copy(ones_v, acc.at[idx.at[pl.ds(j * K, K)]], sem, add=True)

        @pl.loop(0, NBW)
        def _(j):
            pltpu.make_async_copy(ones_v, acc.at[idx.at[pl.ds(j * K, K)]], sem).wait()

        plsc.subcore_barrier()
        pltpu.sync_copy(acc.at[pl.ds(r0, NP // NS)], out_hbm.at[c, pl.ds(r0, NP // NS)])

    return k(zeros, ones, ei)


def _dinv_block(deg_ref):
    deg = deg_ref[0] + deg_ref[1]
    return lax.rsqrt(deg[:, 0:1] + 1.0)


def _l1_body(deg_ref, x_ref, w_ref, o_ref):
    h = jnp.dot(x_ref[...], w_ref[...], preferred_element_type=jnp.float32)
    hs = h * _dinv_block(deg_ref)
    o_ref[0] = hs[:, :HH]
    o_ref[1] = hs[:, HH:]


def _l2_body(deg_ref, a_ref, hs_ref, b_ref, w_ref, o_ref):
    dinv = _dinv_block(deg_ref)
    t = a_ref[...] + hs_ref[...]
    z = jnp.concatenate([t[0], t[1]], axis=1) * dinv + b_ref[...]
    o1 = jnp.maximum(z, 0.0)
    h2 = jnp.dot(o1, w_ref[...], preferred_element_type=jnp.float32)
    o_ref[...] = h2 * dinv


def _out_body(deg_ref, a_ref, hs_ref, b_ref, o_ref):
    dinv = _dinv_block(deg_ref)
    z = (a_ref[0] + a_ref[1] + hs_ref[...]) * dinv + b_ref[...]
    m = jnp.max(z, axis=1, keepdims=True)
    lse = jnp.log(jnp.sum(jnp.exp(z - m), axis=1, keepdims=True)) + m
    o_ref[...] = (z - lse)[:, :C]


def _deg_spec():
    return pl.BlockSpec((2, BM, 16), lambda i: (0, i, 0))


def _row_spec(width):
    return pl.BlockSpec((BM, width), lambda i: (i, 0))


def _agg_spec(width):
    return pl.BlockSpec((2, BM, width), lambda i: (0, i, 0))


def _full_spec(shape):
    return pl.BlockSpec(shape, lambda i: (0,) * len(shape))


def kernel(x, edge_index, W1, b1, W2, b2):
    ei = edge_index.astype(jnp.int32)

    degp = _sc_degree(ei)

    grid = (NP // BM,)
    hs1 = pl.pallas_call(
        _l1_body,
        grid=grid,
        in_specs=[_deg_spec(), _row_spec(D), _full_spec((D, H))],
        out_specs=_agg_spec(HH),
        out_shape=jax.ShapeDtypeStruct((2, NP, HH), jnp.float32),
    )(degp, x, W1)

    a1 = _sc_agg_colsplit(hs1, ei)

    W2p = jnp.pad(W2, ((0, 0), (0, CP - C)))
    b1r = b1.reshape(1, H)
    hs2 = pl.pallas_call(
        _l2_body,
        grid=grid,
        in_specs=[_deg_spec(), _agg_spec(HH), _agg_spec(HH),
                  _full_spec((1, H)), _full_spec((H, CP))],
        out_specs=_row_spec(CP),
        out_shape=jax.ShapeDtypeStruct((NP, CP), jnp.float32),
    )(degp, a1, hs1, b1r, W2p)

    a2 = _sc_segment_sum(hs2, ei, CP, 8)

    b2p = jnp.concatenate([b2, jnp.full((CP - C,), -1e30, jnp.float32)])
    b2r = b2p.reshape(1, CP)
    out = pl.pallas_call(
        _out_body,
        grid=grid,
        in_specs=[_deg_spec(), _agg_spec(CP), _row_spec(CP),
                  _full_spec((1, CP))],
        out_specs=_row_spec(C),
        out_shape=jax.ShapeDtypeStruct((NP, C), jnp.float32),
    )(degp, a2, hs2, b2r)
    return out[:N]


# speedup vs baseline: 1.1020x; 1.0739x over previous
import functools

import jax
import jax.numpy as jnp
from jax import lax
from jax.experimental import pallas as pl
from jax.experimental.pallas import tpu as pltpu
from jax.experimental.pallas import tpu_sc as plsc

N = 10000
NP = 10240
E = 320000
D = 128
H = 128
C = 40
CP = 48

NC = 2
NS = 16
NW = NC * NS
K = 80
EPW = E // NW
RPW = NP // NS

BM = 2560


NBLK = E // K
NBW = NBLK // NW
_MESH = plsc.VectorSubcoreMesh(core_axis_name="c", subcore_axis_name="s")
_SC_PARAMS = pltpu.CompilerParams(use_tc_tiling_on_sc=False)


def _sc_segment_sum(table, ei, width, nbuf):
    nt = (NBW // nbuf) * nbuf
    zeros = jnp.zeros((NP, width), jnp.float32)

    @functools.partial(
        pl.kernel,
        mesh=_MESH,
        compiler_params=_SC_PARAMS,
        out_type=jax.ShapeDtypeStruct((NP, 128), jnp.float32),
        scratch_types=[
            pltpu.VMEM((NBW * K,), jnp.int32),
            pltpu.VMEM((NBW * K,), jnp.int32),
            [pltpu.VMEM((K, width), jnp.float32)] * nbuf,
            pltpu.VMEM_SHARED((NP, width), jnp.float32),
            [pltpu.SemaphoreType.DMA] * nbuf,
        ],
    )
    def k(zeros_hbm, table_hbm, ei_hbm, out_hbm,
          sidx, didx, bufs, acc, sems):
        c = lax.axis_index("c")
        s = lax.axis_index("s")
        wid = c * NS + s
        r0 = s * RPW
        pltpu.sync_copy(zeros_hbm.at[pl.ds(r0, RPW)], acc.at[pl.ds(r0, RPW)])
        pltpu.sync_copy(ei_hbm.at[0, pl.ds(wid * NBW * K, NBW * K)], sidx)
        pltpu.sync_copy(ei_hbm.at[1, pl.ds(wid * NBW * K, NBW * K)], didx)
        plsc.subcore_barrier()

        for b in range(nbuf):
            pltpu.async_copy(table_hbm.at[sidx.at[pl.ds(b * K, K)]], bufs[b], sems[b])

        @pl.loop(0, nt, step=nbuf)
        def _(j):
            for b in range(nbuf):
                pltpu.make_async_copy(
                    table_hbm.at[sidx.at[pl.ds((j + b) * K, K)]], bufs[b], sems[b]).wait()
                pltpu.sync_copy(
                    bufs[b], acc.at[didx.at[pl.ds((j + b) * K, K)]], add=True)

                @pl.when(j + b + nbuf < NBW)
                def _():
                    pltpu.async_copy(
                        table_hbm.at[sidx.at[pl.ds((j + b + nbuf) * K, K)]],
                        bufs[b], sems[b])

        for t in range(nt, NBW):
            b = t % nbuf
            pltpu.make_async_copy(
                table_hbm.at[sidx.at[pl.ds(t * K, K)]], bufs[b], sems[b]).wait()
            pltpu.sync_copy(bufs[b], acc.at[didx.at[pl.ds(t * K, K)]], add=True)

        plsc.subcore_barrier()
        pltpu.sync_copy(acc.at[pl.ds(r0, RPW)],
                        out_hbm.at[pl.ds(r0, RPW), pl.ds(c * width, width)])

    return k(zeros, table, ei)


HH = H // 2
NBS = NBLK // NS
NCHUNK = 2
CBW = NBS // NCHUNK
NB1 = 12


def _sc_agg_colsplit(table2, ei):
    zeros = jnp.zeros((NP, HH), jnp.float32)

    @functools.partial(
        pl.kernel,
        mesh=_MESH,
        compiler_params=_SC_PARAMS,
        out_type=jax.ShapeDtypeStruct((NC, NP, HH), jnp.float32),
        scratch_types=[
            pltpu.VMEM((CBW * K,), jnp.int32),
            pltpu.VMEM((CBW * K,), jnp.int32),
            [pltpu.VMEM((K, HH), jnp.float32)] * NB1,
            pltpu.VMEM_SHARED((NP, HH), jnp.float32),
            [pltpu.SemaphoreType.DMA] * NB1,
        ],
    )
    def k(zeros_hbm, table_hbm, ei_hbm, out_hbm,
          sidx, didx, bufs, acc, sems):
        c = lax.axis_index("c")
        s = lax.axis_index("s")
        r0 = s * RPW
        pltpu.sync_copy(zeros_hbm.at[pl.ds(r0, RPW)], acc.at[pl.ds(r0, RPW)])
        plsc.subcore_barrier()
        half = table_hbm.at[c]

        for chunk in range(NCHUNK):
            base = (s * NBS + chunk * CBW) * K
            pltpu.sync_copy(ei_hbm.at[0, pl.ds(base, CBW * K)], sidx)
            pltpu.sync_copy(ei_hbm.at[1, pl.ds(base, CBW * K)], didx)

            for b in range(NB1):
                pltpu.async_copy(half.at[sidx.at[pl.ds(b * K, K)]], bufs[b], sems[b])

            @pl.loop(0, (CBW // NB1) * NB1, step=NB1)
            def _(j):
                for b in range(NB1):
                    pltpu.make_async_copy(
                        half.at[sidx.at[pl.ds((j + b) * K, K)]], bufs[b], sems[b]).wait()
                    pltpu.sync_copy(
                        bufs[b], acc.at[didx.at[pl.ds((j + b) * K, K)]], add=True)

                    @pl.when(j + b + NB1 < CBW)
                    def _():
                        pltpu.async_copy(
                            half.at[sidx.at[pl.ds((j + b + NB1) * K, K)]],
                            bufs[b], sems[b])

            for t in range((CBW // NB1) * NB1, CBW):
                b = t % NB1
                pltpu.make_async_copy(
                    half.at[sidx.at[pl.ds(t * K, K)]], bufs[b], sems[b]).wait()
                pltpu.sync_copy(bufs[b], acc.at[didx.at[pl.ds(t * K, K)]], add=True)

        plsc.subcore_barrier()
        pltpu.sync_copy(acc.at[pl.ds(r0, RPW)], out_hbm.at[c, pl.ds(r0, RPW)])

    return k(zeros, table2, ei)


def _sc_degree(ei):
    zeros = jnp.zeros((NP, 16), jnp.float32)
    ones = jnp.ones((K, 16), jnp.float32)

    @functools.partial(
        pl.kernel,
        mesh=_MESH,
        compiler_params=_SC_PARAMS,
        out_type=jax.ShapeDtypeStruct((NP, 128), jnp.float32),
        scratch_types=[
            pltpu.VMEM((NBW * K,), jnp.int32),
            pltpu.VMEM((K, 16), jnp.float32),
            pltpu.VMEM_SHARED((NP, 16), jnp.float32),
            pltpu.SemaphoreType.DMA,
        ],
    )
    def k(zeros_hbm, ones_hbm, ei_hbm, out_hbm, idx, ones_v, acc, sem):
        c = lax.axis_index("c")
        s = lax.axis_index("s")
        wid = c * NS + s
        r0 = s * (NP // NS)
        pltpu.sync_copy(zeros_hbm.at[pl.ds(r0, NP // NS)], acc.at[pl.ds(r0, NP // NS)])
        pltpu.sync_copy(ones_hbm, ones_v)
        pltpu.sync_copy(ei_hbm.at[1, pl.ds(wid * NBW * K, NBW * K)], idx)
        plsc.subcore_barrier()

        @pl.loop(0, NBW)
        def _(j):
            pltpu.async_copy(ones_v, acc.at[idx.at[pl.ds(j * K, K)]], sem, add=True)

        @pl.loop(0, NBW)
        def _(j):
            pltpu.make_async_copy(ones_v, acc.at[idx.at[pl.ds(j * K, K)]], sem).wait()

        plsc.subcore_barrier()
        pltpu.sync_copy(acc.at[pl.ds(r0, NP // NS)],
                        out_hbm.at[pl.ds(r0, NP // NS), pl.ds(c * 16, 16)])

    return k(zeros, ones, ei)


def _dinv_block(deg_ref):
    deg = deg_ref[:, 0:1] + deg_ref[:, 16:17]
    return lax.rsqrt(deg + 1.0)


def _l1_body(deg_ref, x_ref, w_ref, o_ref):
    h = jnp.dot(x_ref[...], w_ref[...], preferred_element_type=jnp.float32)
    hs = h * _dinv_block(deg_ref)
    o_ref[0] = hs[:, :HH]
    o_ref[1] = hs[:, HH:]


def _l2_body(deg_ref, a_ref, hs_ref, b_ref, w_ref, o_ref):
    dinv = _dinv_block(deg_ref)
    t = a_ref[...] + hs_ref[...]
    z = jnp.concatenate([t[0], t[1]], axis=1) * dinv + b_ref[...]
    o1 = jnp.maximum(z, 0.0)
    h2 = jnp.dot(o1, w_ref[...], preferred_element_type=jnp.float32)
    o_ref[...] = h2 * dinv


def _out_body(deg_ref, a_ref, hs_ref, b_ref, o_ref):
    dinv = _dinv_block(deg_ref)
    a = a_ref[...]
    z = (a[:, :CP] + a[:, CP:2 * CP] + hs_ref[...]) * dinv + b_ref[...]
    m = jnp.max(z, axis=1, keepdims=True)
    lse = jnp.log(jnp.sum(jnp.exp(z - m), axis=1, keepdims=True)) + m
    o_ref[...] = (z - lse)[:, :C]


def _deg_spec():
    return pl.BlockSpec((BM, 128), lambda i: (i, 0))


def _row_spec(width):
    return pl.BlockSpec((BM, width), lambda i: (i, 0))


def _agg_spec(width):
    return pl.BlockSpec((2, BM, width), lambda i: (0, i, 0))


def _full_spec(shape):
    return pl.BlockSpec(shape, lambda i: (0,) * len(shape))


def kernel(x, edge_index, W1, b1, W2, b2):
    ei = edge_index.astype(jnp.int32)

    degp = _sc_degree(ei)

    grid = (NP // BM,)
    hs1 = pl.pallas_call(
        _l1_body,
        grid=grid,
        in_specs=[_deg_spec(), _row_spec(D), _full_spec((D, H))],
        out_specs=_agg_spec(HH),
        out_shape=jax.ShapeDtypeStruct((2, NP, HH), jnp.float32),
    )(degp, x, W1)

    a1 = _sc_agg_colsplit(hs1, ei)

    W2p = jnp.pad(W2, ((0, 0), (0, CP - C)))
    b1r = b1.reshape(1, H)
    hs2 = pl.pallas_call(
        _l2_body,
        grid=grid,
        in_specs=[_deg_spec(), _agg_spec(HH), _agg_spec(HH),
                  _full_spec((1, H)), _full_spec((H, CP))],
        out_specs=_row_spec(CP),
        out_shape=jax.ShapeDtypeStruct((NP, CP), jnp.float32),
    )(degp, a1, hs1, b1r, W2p)

    a2 = _sc_segment_sum(hs2, ei, CP, 8)

    b2p = jnp.concatenate([b2, jnp.full((CP - C,), -1e30, jnp.float32)])
    b2r = b2p.reshape(1, CP)
    out = pl.pallas_call(
        _out_body,
        grid=grid,
        in_specs=[_deg_spec(), _row_spec(128), _row_spec(CP),
                  _full_spec((1, CP))],
        out_specs=_row_spec(C),
        out_shape=jax.ShapeDtypeStruct((NP, C), jnp.float32),
    )(degp, a2, hs2, b2r)
    return out[:N]


# speedup vs baseline: 1.1510x; 1.0445x over previous
import functools

import jax
import jax.numpy as jnp
from jax import lax
from jax.experimental import pallas as pl
from jax.experimental.pallas import tpu as pltpu
from jax.experimental.pallas import tpu_sc as plsc

N = 10000
NP = 10240
E = 320000
D = 128
H = 128
C = 40
CP = 48

NC = 2
NS = 16
NW = NC * NS
K = 80
EPW = E // NW
RPW = NP // NS

BM = 2560


NBLK = E // K
NBW = NBLK // NW
_MESH = plsc.VectorSubcoreMesh(core_axis_name="c", subcore_axis_name="s")
_SC_PARAMS = pltpu.CompilerParams(use_tc_tiling_on_sc=False)


def _sc_segment_sum(table, ei, width, nbuf):
    nt = (NBW // nbuf) * nbuf
    zeros = jnp.zeros((NP, width), jnp.float32)

    @functools.partial(
        pl.kernel,
        mesh=_MESH,
        compiler_params=_SC_PARAMS,
        out_type=jax.ShapeDtypeStruct((NP, 128), jnp.float32),
        scratch_types=[
            pltpu.VMEM((NBW * K,), jnp.int32),
            pltpu.VMEM((NBW * K,), jnp.int32),
            [pltpu.VMEM((K, width), jnp.float32)] * nbuf,
            pltpu.VMEM_SHARED((NP, width), jnp.float32),
            [pltpu.SemaphoreType.DMA] * nbuf,
        ],
    )
    def k(zeros_hbm, table_hbm, ei_hbm, out_hbm,
          sidx, didx, bufs, acc, sems):
        c = lax.axis_index("c")
        s = lax.axis_index("s")
        wid = c * NS + s
        r0 = s * RPW
        pltpu.sync_copy(zeros_hbm.at[pl.ds(r0, RPW)], acc.at[pl.ds(r0, RPW)])
        pltpu.sync_copy(ei_hbm.at[0, pl.ds(wid * NBW * K, NBW * K)], sidx)
        pltpu.sync_copy(ei_hbm.at[1, pl.ds(wid * NBW * K, NBW * K)], didx)
        plsc.subcore_barrier()

        for b in range(nbuf):
            pltpu.async_copy(table_hbm.at[sidx.at[pl.ds(b * K, K)]], bufs[b], sems[b])

        @pl.loop(0, nt, step=nbuf)
        def _(j):
            for b in range(nbuf):
                pltpu.make_async_copy(
                    table_hbm.at[sidx.at[pl.ds((j + b) * K, K)]], bufs[b], sems[b]).wait()
                pltpu.sync_copy(
                    bufs[b], acc.at[didx.at[pl.ds((j + b) * K, K)]], add=True)

                @pl.when(j + b + nbuf < NBW)
                def _():
                    pltpu.async_copy(
                        table_hbm.at[sidx.at[pl.ds((j + b + nbuf) * K, K)]],
                        bufs[b], sems[b])

        for t in range(nt, NBW):
            b = t % nbuf
            pltpu.make_async_copy(
                table_hbm.at[sidx.at[pl.ds(t * K, K)]], bufs[b], sems[b]).wait()
            pltpu.sync_copy(bufs[b], acc.at[didx.at[pl.ds(t * K, K)]], add=True)

        plsc.subcore_barrier()
        pltpu.sync_copy(acc.at[pl.ds(r0, RPW)],
                        out_hbm.at[pl.ds(r0, RPW), pl.ds(c * width, width)])

    return k(zeros, table, ei)


HH = H // 2
NBS = NBLK // NS
NCHUNK = 2
CBW = NBS // NCHUNK
NB1 = 12


def _sc_agg_colsplit(table2, ei):
    zeros = jnp.zeros((NP, HH), jnp.float32)

    @functools.partial(
        pl.kernel,
        mesh=_MESH,
        compiler_params=_SC_PARAMS,
        out_type=jax.ShapeDtypeStruct((NP, H), jnp.float32),
        scratch_types=[
            pltpu.VMEM((CBW * K,), jnp.int32),
            pltpu.VMEM((CBW * K,), jnp.int32),
            [pltpu.VMEM((K, HH), jnp.float32)] * NB1,
            pltpu.VMEM_SHARED((NP, HH), jnp.float32),
            [pltpu.SemaphoreType.DMA] * NB1,
        ],
    )
    def k(zeros_hbm, table_hbm, ei_hbm, out_hbm,
          sidx, didx, bufs, acc, sems):
        c = lax.axis_index("c")
        s = lax.axis_index("s")
        r0 = s * RPW
        pltpu.sync_copy(zeros_hbm.at[pl.ds(r0, RPW)], acc.at[pl.ds(r0, RPW)])
        plsc.subcore_barrier()
        half = table_hbm.at[c]

        for chunk in range(NCHUNK):
            base = (s * NBS + chunk * CBW) * K
            pltpu.sync_copy(ei_hbm.at[0, pl.ds(base, CBW * K)], sidx)
            pltpu.sync_copy(ei_hbm.at[1, pl.ds(base, CBW * K)], didx)

            for b in range(NB1):
                pltpu.async_copy(half.at[sidx.at[pl.ds(b * K, K)]], bufs[b], sems[b])

            @pl.loop(0, (CBW // NB1) * NB1, step=NB1)
            def _(j):
                for b in range(NB1):
                    pltpu.make_async_copy(
                        half.at[sidx.at[pl.ds((j + b) * K, K)]], bufs[b], sems[b]).wait()
                    pltpu.sync_copy(
                        bufs[b], acc.at[didx.at[pl.ds((j + b) * K, K)]], add=True)

                    @pl.when(j + b + NB1 < CBW)
                    def _():
                        pltpu.async_copy(
                            half.at[sidx.at[pl.ds((j + b + NB1) * K, K)]],
                            bufs[b], sems[b])

            for t in range((CBW // NB1) * NB1, CBW):
                b = t % NB1
                pltpu.make_async_copy(
                    half.at[sidx.at[pl.ds(t * K, K)]], bufs[b], sems[b]).wait()
                pltpu.sync_copy(bufs[b], acc.at[didx.at[pl.ds(t * K, K)]], add=True)

        plsc.subcore_barrier()
        pltpu.sync_copy(acc.at[pl.ds(r0, RPW)],
                        out_hbm.at[pl.ds(r0, RPW), pl.ds(c * HH, HH)])

    return k(zeros, table2, ei)


def _sc_degree(ei):
    zeros = jnp.zeros((NP, 16), jnp.float32)
    ones = jnp.ones((K, 16), jnp.float32)

    @functools.partial(
        pl.kernel,
        mesh=_MESH,
        compiler_params=_SC_PARAMS,
        out_type=jax.ShapeDtypeStruct((NP, 128), jnp.float32),
        scratch_types=[
            pltpu.VMEM((NBW * K,), jnp.int32),
            pltpu.VMEM((K, 16), jnp.float32),
            pltpu.VMEM_SHARED((NP, 16), jnp.float32),
            pltpu.SemaphoreType.DMA,
        ],
    )
    def k(zeros_hbm, ones_hbm, ei_hbm, out_hbm, idx, ones_v, acc, sem):
        c = lax.axis_index("c")
        s = lax.axis_index("s")
        wid = c * NS + s
        r0 = s * (NP // NS)
        pltpu.sync_copy(zeros_hbm.at[pl.ds(r0, NP // NS)], acc.at[pl.ds(r0, NP // NS)])
        pltpu.sync_copy(ones_hbm, ones_v)
        pltpu.sync_copy(ei_hbm.at[1, pl.ds(wid * NBW * K, NBW * K)], idx)
        plsc.subcore_barrier()

        @pl.loop(0, NBW)
        def _(j):
            pltpu.async_copy(ones_v, acc.at[idx.at[pl.ds(j * K, K)]], sem, add=True)

        @pl.loop(0, NBW)
        def _(j):
            pltpu.make_async_copy(ones_v, acc.at[idx.at[pl.ds(j * K, K)]], sem).wait()

        plsc.subcore_barrier()
        pltpu.sync_copy(acc.at[pl.ds(r0, NP // NS)],
                        out_hbm.at[pl.ds(r0, NP // NS), pl.ds(c * 16, 16)])

    return k(zeros, ones, ei)


def _dinv_block(deg_ref):
    deg = deg_ref[:, 0:1] + deg_ref[:, 16:17]
    return lax.rsqrt(deg + 1.0)


def _l1_body(deg_ref, x_ref, w_ref, o_ref):
    h = jnp.dot(x_ref[...], w_ref[...], preferred_element_type=jnp.float32)
    hs = h * _dinv_block(deg_ref)
    o_ref[0] = hs[:, :HH]
    o_ref[1] = hs[:, HH:]


def _l2_body(deg_ref, a_ref, hs_ref, b_ref, w_ref, o_ref):
    dinv = _dinv_block(deg_ref)
    hs = hs_ref[...]
    z = (a_ref[...] + jnp.concatenate([hs[0], hs[1]], axis=1)) * dinv + b_ref[...]
    o1 = jnp.maximum(z, 0.0)
    h2 = jnp.dot(o1, w_ref[...], preferred_element_type=jnp.float32)
    o_ref[...] = h2 * dinv


def _out_body(deg_ref, a_ref, hs_ref, b_ref, o_ref):
    dinv = _dinv_block(deg_ref)
    a = a_ref[...]
    z = (a[:, :CP] + a[:, CP:2 * CP] + hs_ref[...]) * dinv + b_ref[...]
    m = jnp.max(z, axis=1, keepdims=True)
    lse = jnp.log(jnp.sum(jnp.exp(z - m), axis=1, keepdims=True)) + m
    o_ref[...] = (z - lse)[:, :C]


def _deg_spec():
    return pl.BlockSpec((BM, 128), lambda i: (i, 0))


def _row_spec(width):
    return pl.BlockSpec((BM, width), lambda i: (i, 0))


def _agg_spec(width):
    return pl.BlockSpec((2, BM, width), lambda i: (0, i, 0))


def _full_spec(shape):
    return pl.BlockSpec(shape, lambda i: (0,) * len(shape))


def kernel(x, edge_index, W1, b1, W2, b2):
    ei = edge_index.astype(jnp.int32)

    degp = _sc_degree(ei)

    grid = (NP // BM,)
    hs1 = pl.pallas_call(
        _l1_body,
        grid=grid,
        in_specs=[_deg_spec(), _row_spec(D), _full_spec((D, H))],
        out_specs=_agg_spec(HH),
        out_shape=jax.ShapeDtypeStruct((2, NP, HH), jnp.float32),
    )(degp, x, W1)

    a1 = _sc_agg_colsplit(hs1, ei)

    W2p = jnp.pad(W2, ((0, 0), (0, CP - C)))
    b1r = b1.reshape(1, H)
    hs2 = pl.pallas_call(
        _l2_body,
        grid=grid,
        in_specs=[_deg_spec(), _row_spec(128), _agg_spec(HH),
                  _full_spec((1, H)), _full_spec((H, CP))],
        out_specs=_row_spec(CP),
        out_shape=jax.ShapeDtypeStruct((NP, CP), jnp.float32),
    )(degp, a1, hs1, b1r, W2p)

    a2 = _sc_segment_sum(hs2, ei, CP, 8)

    b2p = jnp.concatenate([b2, jnp.full((CP - C,), -1e30, jnp.float32)])
    b2r = b2p.reshape(1, CP)
    out = pl.pallas_call(
        _out_body,
        grid=grid,
        in_specs=[_deg_spec(), _row_spec(128), _row_spec(CP),
                  _full_spec((1, CP))],
        out_specs=_row_spec(C),
        out_shape=jax.ShapeDtypeStruct((NP, C), jnp.float32),
    )(degp, a2, hs2, b2r)
    return out[:N]
